# Initial kernel scaffold; baseline (speedup 1.0000x reference)
import jax
import jax.numpy as jnp
from jax.experimental import pallas as pl


def kernel(x, edge_index, edge_weight, W1, b1, W2, b2, Wp, bp):
    raise NotImplementedError("write your pallas kernel here")


# speedup vs baseline: 1.4854x; 1.4854x over previous
import functools

import jax
import jax.numpy as jnp
from jax import lax
from jax.experimental import pallas as pl
from jax.experimental.pallas import tpu as pltpu
from jax.experimental.pallas import tpu_sc as plsc

_N = 10000
_E = 160000
_DIN = 128
_D = 512
_K = 10
_DELTA = 0.311
_EPS = 1e-3
_TOTVAR = 0.785
_BALANCE = 0.514

_NW = 32
_NS = 16
_BATCH = 32
_NBATCH = 157
_EPAD = _NW * _NBATCH * _BATCH
_BLK = 400
_NBLK = _N // _BLK
_NPAD = 10240
_ROWS_PER_SUB = _NPAD // _NS
_DEGPAD = 10240
_DROWS_PER_SUB = _DEGPAD // _NS
_QIDX = 1001


def _elu(v):
    return jnp.where(v > 0, v, jnp.exp(jnp.minimum(v, 0.0)) - 1.0)


def _mesh():
    return plsc.VectorSubcoreMesh(core_axis_name="c", subcore_axis_name="s",
                                  num_cores=2, num_subcores=16)


def _mm_chunks(x, w, b):
    din = x.shape[1]

    def body(x_ref, w_ref, b_ref, of, *outs):
        y = jnp.dot(x_ref[...], w_ref[...],
                    preferred_element_type=jnp.float32) + b_ref[...]
        of[...] = y
        for t in range(8):
            outs[t][...] = y[:, 64 * t:64 * (t + 1)]

    return pl.pallas_call(
        body,
        grid=(_NBLK,),
        in_specs=[
            pl.BlockSpec((_BLK, din), lambda i: (i, 0)),
            pl.BlockSpec((din, _D), lambda i: (0, 0)),
            pl.BlockSpec((1, _D), lambda i: (0, 0)),
        ],
        out_specs=(pl.BlockSpec((_BLK, _D), lambda i: (i, 0)),) + tuple(
            pl.BlockSpec((_BLK, 64), lambda i: (i, 0)) for _ in range(8)),
        out_shape=(jax.ShapeDtypeStruct((_N, _D), jnp.float32),) + tuple(
            jax.ShapeDtypeStruct((_N, 64), jnp.float32) for _ in range(8)),
    )(x, w, b)


def _gamma_pass(yfull, rowp, colp, wp):
    @functools.partial(
        pl.kernel,
        out_type=jax.ShapeDtypeStruct((_NW, _NBATCH, _BATCH), jnp.float32),
        mesh=_mesh(),
        compiler_params=pltpu.CompilerParams(use_tc_tiling_on_sc=False),
        scratch_types=[
            pltpu.VMEM((_NBATCH, _BATCH), jnp.int32),
            pltpu.VMEM((_NBATCH, _BATCH), jnp.int32),
            pltpu.VMEM((_NBATCH, _BATCH), jnp.float32),
            pltpu.VMEM((_NBATCH, _BATCH), jnp.float32),
            pltpu.VMEM((_BATCH, _D), jnp.float32),
            pltpu.VMEM((_BATCH, _D), jnp.float32),
            pltpu.VMEM((_BATCH, 16), jnp.float32),
        ],
    )
    def k(yf, row_h, col_h, w_h, gamma_h,
          rowv, colv, wv, gb, rb, cb, accbuf):
        wid = lax.axis_index("s") * 2 + lax.axis_index("c")
        pltpu.sync_copy(row_h.at[wid], rowv)
        pltpu.sync_copy(col_h.at[wid], colv)
        pltpu.sync_copy(w_h.at[wid], wv)

        lane16 = lax.iota(jnp.int32, 16)

        def lanesum(a):
            for kk in (1, 2, 4, 8):
                perm = jnp.bitwise_xor(lane16, kk)
                a = a + a.at[perm].get(mode="promise_in_bounds")
            return a

        def batch(j, carry):
            pltpu.sync_copy(yf.at[rowv.at[j]], rb)
            pltpu.sync_copy(yf.at[colv.at[j]], cb)

            def ebody(e, c2):
                def dq(q, acc):
                    return acc + jnp.abs(rb[e, pl.ds(16 * q, 16)]
                                         - cb[e, pl.ds(16 * q, 16)])

                acc = lax.fori_loop(0, _D // 16, dq,
                                    jnp.zeros((16,), jnp.float32),
                                    unroll=4)
                accbuf[e, pl.ds(0, 16)] = acc
                return c2

            lax.fori_loop(0, _BATCH, ebody, 0)
            for g in range(2):
                accv = jnp.zeros((16,), jnp.float32)
                for e16 in range(16):
                    v = accbuf[16 * g + e16, pl.ds(0, 16)]
                    accv = jnp.where(lane16 == e16, lanesum(v), accv)
                w16 = wv[j, pl.ds(16 * g, 16)]
                gb[j, pl.ds(16 * g, 16)] = w16 / jnp.maximum(accv, _EPS)
            return carry

        lax.fori_loop(0, _NBATCH, batch, 0)
        pltpu.sync_copy(gb, gamma_h.at[wid])

    return k(yfull, rowp, colp, wp)


def _seg_pass(y8, rowp, colp, gamma, z2d, z1d):
    @functools.partial(
        pl.kernel,
        out_type=(
            jax.ShapeDtypeStruct((8, _NPAD, 64), jnp.float32),
            jax.ShapeDtypeStruct((8, _NPAD, 64), jnp.float32),
            jax.ShapeDtypeStruct((_DEGPAD,), jnp.float32),
            jax.ShapeDtypeStruct((_DEGPAD,), jnp.float32),
        ),
        mesh=_mesh(),
        compiler_params=pltpu.CompilerParams(use_tc_tiling_on_sc=False),
        scratch_types=[
            pltpu.VMEM((_NBATCH, _BATCH), jnp.int32),
            pltpu.VMEM((_NBATCH, _BATCH), jnp.int32),
            pltpu.VMEM((_NBATCH, _BATCH), jnp.float32),
            pltpu.VMEM((_BATCH, 64), jnp.float32),
            pltpu.VMEM((_BATCH, 64), jnp.float32),
            pltpu.VMEM_SHARED((_NPAD, 64), jnp.float32),
            pltpu.VMEM_SHARED((_DEGPAD,), jnp.float32),
        ],
    )
    def k(y0, y1, y2, y3, y4, y5, y6, y7, row_h, col_h, g_h, z2_h, z1_h,
          agg_a, agg_b, deg_a, deg_b,
          rowv, colv, gv, yb, sb, acc_sp, deg_sp):
        ys = (y0, y1, y2, y3, y4, y5, y6, y7)
        c = lax.axis_index("c")
        s = lax.axis_index("s")
        wid = s * 2 + c
        pltpu.sync_copy(row_h.at[wid], rowv)
        pltpu.sync_copy(col_h.at[wid], colv)
        pltpu.sync_copy(g_h.at[wid], gv)
        r0 = s * _ROWS_PER_SUB
        d0 = s * _DROWS_PER_SUB
        for t in range(8):
            pltpu.sync_copy(z2_h, acc_sp.at[pl.ds(r0, _ROWS_PER_SUB)])
            if t == 0:
                pltpu.sync_copy(z1_h, deg_sp.at[pl.ds(d0, _DROWS_PER_SUB)])
            plsc.subcore_barrier()

            def batch(j, carry):
                pltpu.sync_copy(ys[t].at[colv.at[j]], yb)
                for g in range(2):
                    gvec = gv[j, pl.ds(16 * g, 16)]
                    for e16 in range(16):
                        e = 16 * g + e16
                        ge = gvec[e16]
                        for q in range(4):
                            sb[e, pl.ds(16 * q, 16)] = (
                                yb[e, pl.ds(16 * q, 16)] * ge)
                pltpu.sync_copy(sb, acc_sp.at[rowv.at[j]], add=True)
                if t == 0:
                    pltpu.sync_copy(gv.at[j], deg_sp.at[rowv.at[j]],
                                    add=True)
                return carry

            lax.fori_loop(0, _NBATCH, batch, 0)
            plsc.subcore_barrier()

            @pl.when(c == 0)
            def _():
                pltpu.sync_copy(acc_sp.at[pl.ds(r0, _ROWS_PER_SUB)],
                                agg_a.at[t].at[pl.ds(r0, _ROWS_PER_SUB)])
                if t == 0:
                    pltpu.sync_copy(deg_sp.at[pl.ds(d0, _DROWS_PER_SUB)],
                                    deg_a.at[pl.ds(d0, _DROWS_PER_SUB)])

            @pl.when(c == 1)
            def _():
                pltpu.sync_copy(acc_sp.at[pl.ds(r0, _ROWS_PER_SUB)],
                                agg_b.at[t].at[pl.ds(r0, _ROWS_PER_SUB)])
                if t == 0:
                    pltpu.sync_copy(deg_sp.at[pl.ds(d0, _DROWS_PER_SUB)],
                                    deg_b.at[pl.ds(d0, _DROWS_PER_SUB)])

            plsc.subcore_barrier()

    return k(y8[0], y8[1], y8[2], y8[3], y8[4], y8[5], y8[6], y8[7],
             rowp, colp, gamma, z2d, z1d)


def _combine_mm(yf, deg_a, deg_b, agg_a, agg_b, w, b):
    def body(yf_ref, da, db, aa, ab, w_ref, b_ref, of, *outs):
        deg = da[...] + db[...]
        scale = 1.0 - _DELTA * deg
        av = aa[...] + ab[...]
        agg = jnp.concatenate([av[t] for t in range(8)], axis=1)
        h = _elu(yf_ref[...] * scale + _DELTA * agg)
        y = jnp.dot(h, w_ref[...], preferred_element_type=jnp.float32)
        y = y + b_ref[...]
        of[...] = y
        for t in range(8):
            outs[t][...] = y[:, 64 * t:64 * (t + 1)]

    return pl.pallas_call(
        body,
        grid=(_NBLK,),
        in_specs=(
            [pl.BlockSpec((_BLK, _D), lambda i: (i, 0))]
            + [pl.BlockSpec((_BLK, 1), lambda i: (i, 0)) for _ in range(2)]
            + [pl.BlockSpec((8, _BLK, 64), lambda i: (0, i, 0))
               for _ in range(2)]
            + [pl.BlockSpec((_D, _D), lambda i: (0, 0)),
               pl.BlockSpec((1, _D), lambda i: (0, 0))]),
        out_specs=(pl.BlockSpec((_BLK, _D), lambda i: (i, 0)),) + tuple(
            pl.BlockSpec((_BLK, 64), lambda i: (i, 0)) for _ in range(8)),
        out_shape=(jax.ShapeDtypeStruct((_N, _D), jnp.float32),) + tuple(
            jax.ShapeDtypeStruct((_N, 64), jnp.float32) for _ in range(8)),
    )(yf, deg_a, deg_b, agg_a, agg_b, w, b)


def _combine_softmax(yf, deg_a, deg_b, agg_a, agg_b, wp, bp):
    def body(yf_ref, da, db, aa, ab, w_ref, b_ref, o_ref):
        deg = da[...] + db[...]
        scale = 1.0 - _DELTA * deg
        av = aa[...] + ab[...]
        agg = jnp.concatenate([av[t] for t in range(8)], axis=1)
        h = _elu(yf_ref[...] * scale + _DELTA * agg)
        logits = jnp.dot(h, w_ref[...],
                         preferred_element_type=jnp.float32) + b_ref[...]
        m = jnp.max(logits, axis=1, keepdims=True)
        e = jnp.exp(logits - m)
        o_ref[...] = e / jnp.sum(e, axis=1, keepdims=True)

    return pl.pallas_call(
        body,
        grid=(_NBLK,),
        in_specs=(
            [pl.BlockSpec((_BLK, _D), lambda i: (i, 0))]
            + [pl.BlockSpec((_BLK, 1), lambda i: (i, 0)) for _ in range(2)]
            + [pl.BlockSpec((8, _BLK, 64), lambda i: (0, i, 0))
               for _ in range(2)]
            + [pl.BlockSpec((_D, 16), lambda i: (0, 0)),
               pl.BlockSpec((1, 16), lambda i: (0, 0))]),
        out_specs=pl.BlockSpec((_BLK, 16), lambda i: (i, 0)),
        out_shape=jax.ShapeDtypeStruct((_N, 16), jnp.float32),
    )(yf, deg_a, deg_b, agg_a, agg_b, wp, bp)


def _tv_pass(s16, rowp, colp, wp):
    @functools.partial(
        pl.kernel,
        out_type=jax.ShapeDtypeStruct((_NW, 16), jnp.float32),
        mesh=_mesh(),
        compiler_params=pltpu.CompilerParams(use_tc_tiling_on_sc=False),
        scratch_types=[
            pltpu.VMEM((_NBATCH, _BATCH), jnp.int32),
            pltpu.VMEM((_NBATCH, _BATCH), jnp.int32),
            pltpu.VMEM((_NBATCH, _BATCH), jnp.float32),
            pltpu.VMEM((_BATCH, 16), jnp.float32),
            pltpu.VMEM((_BATCH, 16), jnp.float32),
            pltpu.VMEM((16,), jnp.float32),
        ],
    )
    def k(s_h, row_h, col_h, w_h, tvp_h,
          rowv, colv, wv, rsb, csb, tvb):
        wid = lax.axis_index("s") * 2 + lax.axis_index("c")
        pltpu.sync_copy(row_h.at[wid], rowv)
        pltpu.sync_copy(col_h.at[wid], colv)
        pltpu.sync_copy(w_h.at[wid], wv)

        def batch(j, acc):
            pltpu.sync_copy(s_h.at[rowv.at[j]], rsb)
            pltpu.sync_copy(s_h.at[colv.at[j]], csb)
            for g in range(2):
                wvec = wv[j, pl.ds(16 * g, 16)]
                for e16 in range(16):
                    e = 16 * g + e16
                    d = jnp.abs(rsb[e, pl.ds(0, 16)]
                                - csb[e, pl.ds(0, 16)])
                    acc = acc + d * wvec[e16]
            return acc

        acc = lax.fori_loop(0, _NBATCH, batch,
                            jnp.zeros((16,), jnp.float32))
        tvb[...] = acc
        pltpu.sync_copy(tvb, tvp_h.at[wid])

    return k(s16, rowp, colp, wp)


def _loss_pass(s16, tvp):
    def body(s_ref, tvp_ref, asym_ref, tvsum_ref):
        sv = s_ref[...]
        lo = jnp.zeros((1, 16), jnp.int32)
        hi = jnp.full((1, 16), 0x3F800000, jnp.int32)

        def bit(i, lh):
            lo_, hi_ = lh
            mid = lo_ + (hi_ - lo_ + 1) // 2
            thr = lax.bitcast_convert_type(mid, jnp.float32)
            cnt = jnp.sum((sv >= thr).astype(jnp.int32), axis=0,
                          keepdims=True)
            ok = cnt >= _QIDX
            return (jnp.where(ok, mid, lo_),
                    jnp.where(ok, hi_, mid - 1))

        lo, hi = lax.fori_loop(0, 31, bit, (lo, hi))
        med = lax.bitcast_convert_type(lo, jnp.float32)
        diff = sv - med
        asym = jnp.sum(jnp.where(diff >= 0, _K - 1.0, 1.0) * jnp.abs(diff))
        asym_ref[...] = jnp.reshape(asym, (1, 1))
        tvsum_ref[...] = jnp.reshape(jnp.sum(tvp_ref[...]), (1, 1))

    return pl.pallas_call(
        body,
        in_specs=[pl.BlockSpec((_N, 16), lambda: (0, 0)),
                  pl.BlockSpec((_NW, 16), lambda: (0, 0))],
        out_specs=(pl.BlockSpec((1, 1), lambda: (0, 0)),
                   pl.BlockSpec((1, 1), lambda: (0, 0))),
        out_shape=(jax.ShapeDtypeStruct((1, 1), jnp.float32),
                   jax.ShapeDtypeStruct((1, 1), jnp.float32)),
    )(s16, tvp)


def kernel(x, edge_index, edge_weight, W1, b1, W2, b2, Wp, bp):
    pad = _EPAD - _E
    row = edge_index[0]
    col = edge_index[1]
    rowp = jnp.concatenate(
        [row, jnp.zeros((pad,), row.dtype)]).reshape(_NW, _NBATCH, _BATCH)
    colp = jnp.concatenate(
        [col, jnp.zeros((pad,), col.dtype)]).reshape(_NW, _NBATCH, _BATCH)
    wpe = jnp.concatenate(
        [edge_weight,
         jnp.zeros((pad,), jnp.float32)]).reshape(_NW, _NBATCH, _BATCH)
    b1r = b1.reshape(1, _D)
    b2r = b2.reshape(1, _D)
    wp_pad = jnp.concatenate([Wp, jnp.zeros((_D, 16 - _K), jnp.float32)],
                             axis=1)
    bp_pad = jnp.concatenate(
        [bp, jnp.full((16 - _K,), -1e30, jnp.float32)]).reshape(1, 16)
    z2d = jnp.zeros((_ROWS_PER_SUB, 64), jnp.float32)
    z1d = jnp.zeros((_DROWS_PER_SUB,), jnp.float32)

    y1f, *y1 = _mm_chunks(x, W1, b1r)
    g1 = _gamma_pass(y1f, rowp, colp, wpe)
    agg_a1, agg_b1, deg_a1, deg_b1 = _seg_pass(y1, rowp, colp, g1, z2d, z1d)
    da1 = deg_a1[:_N].reshape(_N, 1)
    db1 = deg_b1[:_N].reshape(_N, 1)
    y2f, *y2 = _combine_mm(y1f, da1, db1, agg_a1, agg_b1, W2, b2r)

    g2 = _gamma_pass(y2f, rowp, colp, wpe)
    agg_a2, agg_b2, deg_a2, deg_b2 = _seg_pass(y2, rowp, colp, g2, z2d, z1d)
    da2 = deg_a2[:_N].reshape(_N, 1)
    db2 = deg_b2[:_N].reshape(_N, 1)
    s16 = _combine_softmax(y2f, da2, db2, agg_a2, agg_b2, wp_pad, bp_pad)

    tvp = _tv_pass(s16, rowp, colp, wpe)
    asym, tvsum = _loss_pass(s16, tvp)

    s_out = s16[:, :_K]
    tv_loss = (_TOTVAR / (2.0 * _E)) * tvsum[0, 0]
    denom = _N * (_K - 1.0)
    bal_loss = _BALANCE * ((denom - asym[0, 0]) / denom)
    return s_out, tv_loss, bal_loss


# speedup vs baseline: 1.6264x; 1.0949x over previous
import functools

import jax
import jax.numpy as jnp
from jax import lax
from jax.experimental import pallas as pl
from jax.experimental.pallas import tpu as pltpu
from jax.experimental.pallas import tpu_sc as plsc

_N = 10000
_E = 160000
_DIN = 128
_D = 512
_K = 10
_DELTA = 0.311
_EPS = 1e-3
_TOTVAR = 0.785
_BALANCE = 0.514

_NW = 32
_NS = 16
_BATCH = 32
_NBATCH = 158
_EPAD = _NW * _NBATCH * _BATCH
_BLK = 400
_NBLK = _N // _BLK
_NPAD = 10240
_ROWS_PER_SUB = _NPAD // _NS
_DEGPAD = 10240
_DROWS_PER_SUB = _DEGPAD // _NS
_QIDX = 1001


def _elu(v):
    return jnp.where(v > 0, v, jnp.exp(jnp.minimum(v, 0.0)) - 1.0)


def _mesh():
    return plsc.VectorSubcoreMesh(core_axis_name="c", subcore_axis_name="s",
                                  num_cores=2, num_subcores=16)


def _mm_chunks(x, w, b):
    din = x.shape[1]

    def body(x_ref, w_ref, b_ref, of, *outs):
        y = jnp.dot(x_ref[...], w_ref[...],
                    preferred_element_type=jnp.float32) + b_ref[...]
        of[...] = y
        for t in range(8):
            outs[t][...] = y[:, 64 * t:64 * (t + 1)]

    return pl.pallas_call(
        body,
        grid=(_NBLK,),
        in_specs=[
            pl.BlockSpec((_BLK, din), lambda i: (i, 0)),
            pl.BlockSpec((din, _D), lambda i: (0, 0)),
            pl.BlockSpec((1, _D), lambda i: (0, 0)),
        ],
        out_specs=(pl.BlockSpec((_BLK, _D), lambda i: (i, 0)),) + tuple(
            pl.BlockSpec((_BLK, 64), lambda i: (i, 0)) for _ in range(8)),
        out_shape=(jax.ShapeDtypeStruct((_N, _D), jnp.float32),) + tuple(
            jax.ShapeDtypeStruct((_N, 64), jnp.float32) for _ in range(8)),
    )(x, w, b)


def _gamma_pass(yfull, rowp, colp, wp):
    @functools.partial(
        pl.kernel,
        out_type=jax.ShapeDtypeStruct((_NW, _NBATCH, _BATCH), jnp.float32),
        mesh=_mesh(),
        compiler_params=pltpu.CompilerParams(use_tc_tiling_on_sc=False),
        scratch_types=[
            pltpu.VMEM((_NBATCH, _BATCH), jnp.int32),
            pltpu.VMEM((_NBATCH, _BATCH), jnp.int32),
            pltpu.VMEM((_NBATCH, _BATCH), jnp.float32),
            pltpu.VMEM((_NBATCH, _BATCH), jnp.float32),
            pltpu.VMEM((16, _D), jnp.float32),
            pltpu.VMEM((16, _D), jnp.float32),
            pltpu.VMEM((16, _D), jnp.float32),
            pltpu.VMEM((16, _D), jnp.float32),
            pltpu.VMEM((16, 16), jnp.float32),
            pltpu.SemaphoreType.DMA,
            pltpu.SemaphoreType.DMA,
        ],
    )
    def k(yf, row_h, col_h, w_h, gamma_h,
          rowv, colv, wv, gb, rb0, cb0, rb1, cb1, accbuf, sem0, sem1):
        wid = lax.axis_index("s") * 2 + lax.axis_index("c")
        pltpu.sync_copy(row_h.at[wid], rowv)
        pltpu.sync_copy(col_h.at[wid], colv)
        pltpu.sync_copy(w_h.at[wid], wv)

        lane16 = lax.iota(jnp.int32, 16)

        def lanesum(a):
            for kk in (1, 2, 4, 8):
                perm = jnp.bitwise_xor(lane16, kk)
                a = a + a.at[perm].get(mode="promise_in_bounds")
            return a

        def issue(j, g, rb, cb, sem):
            pltpu.async_copy(yf.at[rowv.at[j].at[pl.ds(16 * g, 16)]],
                             rb, sem)
            pltpu.async_copy(yf.at[colv.at[j].at[pl.ds(16 * g, 16)]],
                             cb, sem)

        def drain(rb, cb, sem):
            pltpu.make_async_copy(yf.at[rowv.at[0].at[pl.ds(0, 16)]],
                                  rb, sem).wait()
            pltpu.make_async_copy(yf.at[colv.at[0].at[pl.ds(0, 16)]],
                                  cb, sem).wait()

        def compute(j, g, rb, cb):
            def ebody(e, c2):
                def dq(q, acc):
                    return acc + jnp.abs(rb[e, pl.ds(16 * q, 16)]
                                         - cb[e, pl.ds(16 * q, 16)])

                acc = lax.fori_loop(0, _D // 16, dq,
                                    jnp.zeros((16,), jnp.float32),
                                    unroll=4)
                accbuf[e, pl.ds(0, 16)] = acc
                return c2

            lax.fori_loop(0, 16, ebody, 0)
            accv = jnp.zeros((16,), jnp.float32)
            for e16 in range(16):
                v = accbuf[e16, pl.ds(0, 16)]
                accv = jnp.where(lane16 == e16, lanesum(v), accv)
            w16 = wv[j, pl.ds(16 * g, 16)]
            gb[j, pl.ds(16 * g, 16)] = w16 / jnp.maximum(accv, _EPS)

        issue(0, 0, rb0, cb0, sem0)

        def batch(j, carry):
            jn = jnp.minimum(j + 1, _NBATCH - 1)
            drain(rb0, cb0, sem0)
            issue(j, 1, rb1, cb1, sem1)
            compute(j, 0, rb0, cb0)
            drain(rb1, cb1, sem1)
            issue(jn, 0, rb0, cb0, sem0)
            compute(j, 1, rb1, cb1)
            return carry

        lax.fori_loop(0, _NBATCH, batch, 0)
        drain(rb0, cb0, sem0)
        pltpu.sync_copy(gb, gamma_h.at[wid])

    return k(yfull, rowp, colp, wp)


def _seg_pass(y8, rowp, colp, gamma, z2d, z1d):
    @functools.partial(
        pl.kernel,
        out_type=(
            jax.ShapeDtypeStruct((8, _NPAD, 64), jnp.float32),
            jax.ShapeDtypeStruct((8, _NPAD, 64), jnp.float32),
            jax.ShapeDtypeStruct((_DEGPAD,), jnp.float32),
            jax.ShapeDtypeStruct((_DEGPAD,), jnp.float32),
        ),
        mesh=_mesh(),
        compiler_params=pltpu.CompilerParams(use_tc_tiling_on_sc=False),
        scratch_types=[
            pltpu.VMEM((_NBATCH, _BATCH), jnp.int32),
            pltpu.VMEM((_NBATCH, _BATCH), jnp.int32),
            pltpu.VMEM((_NBATCH, _BATCH), jnp.float32),
            pltpu.VMEM((_BATCH, 64), jnp.float32),
            pltpu.VMEM((_BATCH, 64), jnp.float32),
            pltpu.VMEM((_BATCH, 64), jnp.float32),
            pltpu.VMEM((_BATCH, 64), jnp.float32),
            pltpu.VMEM_SHARED((_NPAD, 64), jnp.float32),
            pltpu.VMEM_SHARED((_DEGPAD,), jnp.float32),
            pltpu.SemaphoreType.DMA,
            pltpu.SemaphoreType.DMA,
            pltpu.SemaphoreType.DMA,
            pltpu.SemaphoreType.DMA,
        ],
    )
    def k(y0, y1, y2, y3, y4, y5, y6, y7, row_h, col_h, g_h, z2_h, z1_h,
          agg_a, agg_b, deg_a, deg_b,
          rowv, colv, gv, yb0, yb1, sb0, sb1, acc_sp, deg_sp,
          gsem0, gsem1, ssem0, ssem1):
        ys = (y0, y1, y2, y3, y4, y5, y6, y7)
        c = lax.axis_index("c")
        s = lax.axis_index("s")
        wid = s * 2 + c
        pltpu.sync_copy(row_h.at[wid], rowv)
        pltpu.sync_copy(col_h.at[wid], colv)
        pltpu.sync_copy(g_h.at[wid], gv)
        r0 = s * _ROWS_PER_SUB
        d0 = s * _DROWS_PER_SUB

        def scale(j, yb, sb):
            for g in range(2):
                gvec = gv[j, pl.ds(16 * g, 16)]
                for e16 in range(16):
                    e = 16 * g + e16
                    ge = gvec[e16]
                    for q in range(4):
                        sb[e, pl.ds(16 * q, 16)] = (
                            yb[e, pl.ds(16 * q, 16)] * ge)

        for t in range(8):
            pltpu.sync_copy(z2_h, acc_sp.at[pl.ds(r0, _ROWS_PER_SUB)])
            if t == 0:
                pltpu.sync_copy(z1_h, deg_sp.at[pl.ds(d0, _DROWS_PER_SUB)])
            plsc.subcore_barrier()

            pltpu.async_copy(ys[t].at[colv.at[0]], yb0, gsem0)

            def pair(j2, carry):
                j = 2 * j2
                jn = j + 1
                jnn = jnp.minimum(j + 2, _NBATCH - 1)
                pltpu.make_async_copy(ys[t].at[colv.at[j]], yb0,
                                      gsem0).wait()
                pltpu.async_copy(ys[t].at[colv.at[jn]], yb1, gsem1)

                @pl.when(j2 > 0)
                def _():
                    pltpu.make_async_copy(sb0, acc_sp.at[rowv.at[j]],
                                          ssem0).wait()

                scale(j, yb0, sb0)
                pltpu.async_copy(sb0, acc_sp.at[rowv.at[j]], ssem0,
                                 add=True)
                if t == 0:
                    pltpu.sync_copy(gv.at[j], deg_sp.at[rowv.at[j]],
                                    add=True)
                pltpu.make_async_copy(ys[t].at[colv.at[jn]], yb1,
                                      gsem1).wait()
                pltpu.async_copy(ys[t].at[colv.at[jnn]], yb0, gsem0)

                @pl.when(j2 > 0)
                def _():
                    pltpu.make_async_copy(sb1, acc_sp.at[rowv.at[jn]],
                                          ssem1).wait()

                scale(jn, yb1, sb1)
                pltpu.async_copy(sb1, acc_sp.at[rowv.at[jn]], ssem1,
                                 add=True)
                if t == 0:
                    pltpu.sync_copy(gv.at[jn], deg_sp.at[rowv.at[jn]],
                                    add=True)
                return carry

            lax.fori_loop(0, _NBATCH // 2, pair, 0)
            pltpu.make_async_copy(sb0, acc_sp.at[rowv.at[0]],
                                  ssem0).wait()
            pltpu.make_async_copy(sb1, acc_sp.at[rowv.at[0]],
                                  ssem1).wait()
            pltpu.make_async_copy(ys[t].at[colv.at[0]], yb0,
                                  gsem0).wait()
            plsc.subcore_barrier()

            @pl.when(c == 0)
            def _():
                pltpu.sync_copy(acc_sp.at[pl.ds(r0, _ROWS_PER_SUB)],
                                agg_a.at[t].at[pl.ds(r0, _ROWS_PER_SUB)])
                if t == 0:
                    pltpu.sync_copy(deg_sp.at[pl.ds(d0, _DROWS_PER_SUB)],
                                    deg_a.at[pl.ds(d0, _DROWS_PER_SUB)])

            @pl.when(c == 1)
            def _():
                pltpu.sync_copy(acc_sp.at[pl.ds(r0, _ROWS_PER_SUB)],
                                agg_b.at[t].at[pl.ds(r0, _ROWS_PER_SUB)])
                if t == 0:
                    pltpu.sync_copy(deg_sp.at[pl.ds(d0, _DROWS_PER_SUB)],
                                    deg_b.at[pl.ds(d0, _DROWS_PER_SUB)])

            plsc.subcore_barrier()

    return k(y8[0], y8[1], y8[2], y8[3], y8[4], y8[5], y8[6], y8[7],
             rowp, colp, gamma, z2d, z1d)


def _combine_mm(yf, deg_a, deg_b, agg_a, agg_b, w, b):
    def body(yf_ref, da, db, aa, ab, w_ref, b_ref, of, *outs):
        deg = da[...] + db[...]
        scale = 1.0 - _DELTA * deg
        av = aa[...] + ab[...]
        agg = jnp.concatenate([av[t] for t in range(8)], axis=1)
        h = _elu(yf_ref[...] * scale + _DELTA * agg)
        y = jnp.dot(h, w_ref[...], preferred_element_type=jnp.float32)
        y = y + b_ref[...]
        of[...] = y
        for t in range(8):
            outs[t][...] = y[:, 64 * t:64 * (t + 1)]

    return pl.pallas_call(
        body,
        grid=(_NBLK,),
        in_specs=(
            [pl.BlockSpec((_BLK, _D), lambda i: (i, 0))]
            + [pl.BlockSpec((_BLK, 1), lambda i: (i, 0)) for _ in range(2)]
            + [pl.BlockSpec((8, _BLK, 64), lambda i: (0, i, 0))
               for _ in range(2)]
            + [pl.BlockSpec((_D, _D), lambda i: (0, 0)),
               pl.BlockSpec((1, _D), lambda i: (0, 0))]),
        out_specs=(pl.BlockSpec((_BLK, _D), lambda i: (i, 0)),) + tuple(
            pl.BlockSpec((_BLK, 64), lambda i: (i, 0)) for _ in range(8)),
        out_shape=(jax.ShapeDtypeStruct((_N, _D), jnp.float32),) + tuple(
            jax.ShapeDtypeStruct((_N, 64), jnp.float32) for _ in range(8)),
    )(yf, deg_a, deg_b, agg_a, agg_b, w, b)


def _combine_softmax(yf, deg_a, deg_b, agg_a, agg_b, wp, bp):
    def body(yf_ref, da, db, aa, ab, w_ref, b_ref, o_ref):
        deg = da[...] + db[...]
        scale = 1.0 - _DELTA * deg
        av = aa[...] + ab[...]
        agg = jnp.concatenate([av[t] for t in range(8)], axis=1)
        h = _elu(yf_ref[...] * scale + _DELTA * agg)
        logits = jnp.dot(h, w_ref[...],
                         preferred_element_type=jnp.float32) + b_ref[...]
        m = jnp.max(logits, axis=1, keepdims=True)
        e = jnp.exp(logits - m)
        o_ref[...] = e / jnp.sum(e, axis=1, keepdims=True)

    return pl.pallas_call(
        body,
        grid=(_NBLK,),
        in_specs=(
            [pl.BlockSpec((_BLK, _D), lambda i: (i, 0))]
            + [pl.BlockSpec((_BLK, 1), lambda i: (i, 0)) for _ in range(2)]
            + [pl.BlockSpec((8, _BLK, 64), lambda i: (0, i, 0))
               for _ in range(2)]
            + [pl.BlockSpec((_D, 16), lambda i: (0, 0)),
               pl.BlockSpec((1, 16), lambda i: (0, 0))]),
        out_specs=pl.BlockSpec((_BLK, 16), lambda i: (i, 0)),
        out_shape=jax.ShapeDtypeStruct((_N, 16), jnp.float32),
    )(yf, deg_a, deg_b, agg_a, agg_b, wp, bp)


def _tv_pass(s16, rowp, colp, wp):
    @functools.partial(
        pl.kernel,
        out_type=jax.ShapeDtypeStruct((_NW, 16), jnp.float32),
        mesh=_mesh(),
        compiler_params=pltpu.CompilerParams(use_tc_tiling_on_sc=False),
        scratch_types=[
            pltpu.VMEM((_NBATCH, _BATCH), jnp.int32),
            pltpu.VMEM((_NBATCH, _BATCH), jnp.int32),
            pltpu.VMEM((_NBATCH, _BATCH), jnp.float32),
            pltpu.VMEM((_BATCH, 16), jnp.float32),
            pltpu.VMEM((_BATCH, 16), jnp.float32),
            pltpu.VMEM((16,), jnp.float32),
        ],
    )
    def k(s_h, row_h, col_h, w_h, tvp_h,
          rowv, colv, wv, rsb, csb, tvb):
        wid = lax.axis_index("s") * 2 + lax.axis_index("c")
        pltpu.sync_copy(row_h.at[wid], rowv)
        pltpu.sync_copy(col_h.at[wid], colv)
        pltpu.sync_copy(w_h.at[wid], wv)

        def batch(j, acc):
            pltpu.sync_copy(s_h.at[rowv.at[j]], rsb)
            pltpu.sync_copy(s_h.at[colv.at[j]], csb)
            for g in range(2):
                wvec = wv[j, pl.ds(16 * g, 16)]
                for e16 in range(16):
                    e = 16 * g + e16
                    d = jnp.abs(rsb[e, pl.ds(0, 16)]
                                - csb[e, pl.ds(0, 16)])
                    acc = acc + d * wvec[e16]
            return acc

        acc = lax.fori_loop(0, _NBATCH, batch,
                            jnp.zeros((16,), jnp.float32))
        tvb[...] = acc
        pltpu.sync_copy(tvb, tvp_h.at[wid])

    return k(s16, rowp, colp, wp)


def _loss_pass(s16, tvp):
    def body(s_ref, tvp_ref, asym_ref, tvsum_ref):
        sv = s_ref[...]
        lo = jnp.zeros((1, 16), jnp.int32)
        hi = jnp.full((1, 16), 0x3F800000, jnp.int32)

        def bit(i, lh):
            lo_, hi_ = lh
            mid = lo_ + (hi_ - lo_ + 1) // 2
            thr = lax.bitcast_convert_type(mid, jnp.float32)
            cnt = jnp.sum((sv >= thr).astype(jnp.int32), axis=0,
                          keepdims=True)
            ok = cnt >= _QIDX
            return (jnp.where(ok, mid, lo_),
                    jnp.where(ok, hi_, mid - 1))

        lo, hi = lax.fori_loop(0, 31, bit, (lo, hi))
        med = lax.bitcast_convert_type(lo, jnp.float32)
        diff = sv - med
        asym = jnp.sum(jnp.where(diff >= 0, _K - 1.0, 1.0) * jnp.abs(diff))
        asym_ref[...] = jnp.reshape(asym, (1, 1))
        tvsum_ref[...] = jnp.reshape(jnp.sum(tvp_ref[...]), (1, 1))

    return pl.pallas_call(
        body,
        in_specs=[pl.BlockSpec((_N, 16), lambda: (0, 0)),
                  pl.BlockSpec((_NW, 16), lambda: (0, 0))],
        out_specs=(pl.BlockSpec((1, 1), lambda: (0, 0)),
                   pl.BlockSpec((1, 1), lambda: (0, 0))),
        out_shape=(jax.ShapeDtypeStruct((1, 1), jnp.float32),
                   jax.ShapeDtypeStruct((1, 1), jnp.float32)),
    )(s16, tvp)


def kernel(x, edge_index, edge_weight, W1, b1, W2, b2, Wp, bp):
    pad = _EPAD - _E
    row = edge_index[0]
    col = edge_index[1]
    rowp = jnp.concatenate(
        [row, jnp.zeros((pad,), row.dtype)]).reshape(_NW, _NBATCH, _BATCH)
    colp = jnp.concatenate(
        [col, jnp.zeros((pad,), col.dtype)]).reshape(_NW, _NBATCH, _BATCH)
    wpe = jnp.concatenate(
        [edge_weight,
         jnp.zeros((pad,), jnp.float32)]).reshape(_NW, _NBATCH, _BATCH)
    b1r = b1.reshape(1, _D)
    b2r = b2.reshape(1, _D)
    wp_pad = jnp.concatenate([Wp, jnp.zeros((_D, 16 - _K), jnp.float32)],
                             axis=1)
    bp_pad = jnp.concatenate(
        [bp, jnp.full((16 - _K,), -1e30, jnp.float32)]).reshape(1, 16)
    z2d = jnp.zeros((_ROWS_PER_SUB, 64), jnp.float32)
    z1d = jnp.zeros((_DROWS_PER_SUB,), jnp.float32)

    y1f, *y1 = _mm_chunks(x, W1, b1r)
    g1 = _gamma_pass(y1f, rowp, colp, wpe)
    agg_a1, agg_b1, deg_a1, deg_b1 = _seg_pass(y1, rowp, colp, g1, z2d, z1d)
    da1 = deg_a1[:_N].reshape(_N, 1)
    db1 = deg_b1[:_N].reshape(_N, 1)
    y2f, *y2 = _combine_mm(y1f, da1, db1, agg_a1, agg_b1, W2, b2r)

    g2 = _gamma_pass(y2f, rowp, colp, wpe)
    agg_a2, agg_b2, deg_a2, deg_b2 = _seg_pass(y2, rowp, colp, g2, z2d, z1d)
    da2 = deg_a2[:_N].reshape(_N, 1)
    db2 = deg_b2[:_N].reshape(_N, 1)
    s16 = _combine_softmax(y2f, da2, db2, agg_a2, agg_b2, wp_pad, bp_pad)

    tvp = _tv_pass(s16, rowp, colp, wpe)
    asym, tvsum = _loss_pass(s16, tvp)

    s_out = s16[:, :_K]
    tv_loss = (_TOTVAR / (2.0 * _E)) * tvsum[0, 0]
    denom = _N * (_K - 1.0)
    bal_loss = _BALANCE * ((denom - asym[0, 0]) / denom)
    return s_out, tv_loss, bal_loss
